# Initial kernel scaffold; baseline (speedup 1.0000x reference)
import jax
import jax.numpy as jnp
from jax.experimental import pallas as pl


def kernel(x, table):
    raise NotImplementedError("write your pallas kernel here")


# speedup vs baseline: 1.3063x; 1.3063x over previous
import functools

import jax
import jax.numpy as jnp
from jax import lax
from jax.experimental import pallas as pl
from jax.experimental.pallas import tpu as pltpu
from jax.experimental.pallas import tpu_sc as plsc

EMBED_DIM = 32
NUM_CORES = 2
NUM_SUBCORES = 16
NUM_WORKERS = NUM_CORES * NUM_SUBCORES
GROUP = 128

_mesh = plsc.VectorSubcoreMesh(core_axis_name="c", subcore_axis_name="s")


@functools.partial(jax.jit, static_argnames=("groups_per_w",))
def _embed_lookup(xg, table, groups_per_w):
    n_rows = NUM_WORKERS * groups_per_w * GROUP

    @functools.partial(
        pl.kernel,
        mesh=_mesh,
        out_type=jax.ShapeDtypeStruct((n_rows, EMBED_DIM), jnp.float32),
        scratch_types=[
            pltpu.VMEM((groups_per_w, GROUP), jnp.int32),
            pltpu.VMEM((GROUP, EMBED_DIM), jnp.float32),
            pltpu.SemaphoreType.DMA,
        ],
        compiler_params=pltpu.CompilerParams(use_tc_tiling_on_sc=False),
    )
    def body(x_hbm, table_hbm, out_hbm, idx_v, rows_v, sem):
        wid = lax.axis_index("s") * NUM_CORES + lax.axis_index("c")
        pltpu.sync_copy(x_hbm.at[wid], idx_v)
        base = wid * (groups_per_w * GROUP)

        def step(j, carry):
            pltpu.async_copy(table_hbm.at[idx_v.at[j]], rows_v, sem).wait()
            pltpu.sync_copy(rows_v, out_hbm.at[pl.ds(base + j * GROUP, GROUP)])
            return carry

        lax.fori_loop(0, groups_per_w, step, 0)

    return body(xg, table)


def kernel(x, table):
    batch, hist = x.shape
    total = batch * hist
    groups_per_w = total // (NUM_WORKERS * GROUP)
    xg = x.reshape(NUM_WORKERS, groups_per_w, GROUP).astype(jnp.int32)
    out = _embed_lookup(xg, table, groups_per_w)
    return out.reshape(batch, hist, EMBED_DIM)


# speedup vs baseline: 1.4982x; 1.1469x over previous
import functools

import jax
import jax.numpy as jnp
from jax import lax
from jax.experimental import pallas as pl
from jax.experimental.pallas import tpu as pltpu
from jax.experimental.pallas import tpu_sc as plsc

EMBED_DIM = 32
NUM_CORES = 2
NUM_SUBCORES = 16
NUM_WORKERS = NUM_CORES * NUM_SUBCORES
GROUP = 128
NBUF = 8
DEPTH = 4

_mesh = plsc.VectorSubcoreMesh(core_axis_name="c", subcore_axis_name="s")


@functools.partial(jax.jit, static_argnames=("groups_per_w",))
def _embed_lookup(xg, table, groups_per_w):
    n_rows = NUM_WORKERS * groups_per_w * GROUP

    @functools.partial(
        pl.kernel,
        mesh=_mesh,
        out_type=jax.ShapeDtypeStruct((n_rows, EMBED_DIM), jnp.float32),
        scratch_types=[
            pltpu.VMEM((groups_per_w, GROUP), jnp.int32),
            pltpu.VMEM((NBUF, GROUP, EMBED_DIM), jnp.float32),
            pltpu.SemaphoreType.DMA((NBUF,)),
            pltpu.SemaphoreType.DMA((NBUF,)),
        ],
        compiler_params=pltpu.CompilerParams(use_tc_tiling_on_sc=False),
    )
    def body(x_hbm, table_hbm, out_hbm, idx_v, rows_v, gsem, wsem):
        wid = lax.axis_index("s") * NUM_CORES + lax.axis_index("c")
        pltpu.sync_copy(x_hbm.at[wid], idx_v)
        base = wid * (groups_per_w * GROUP)

        def gather(j, slot):
            return pltpu.make_async_copy(
                table_hbm.at[idx_v.at[j]], rows_v.at[slot], gsem.at[slot])

        def write(j, slot):
            return pltpu.make_async_copy(
                rows_v.at[slot], out_hbm.at[pl.ds(base + j * GROUP, GROUP)],
                wsem.at[slot])

        for b in range(DEPTH):
            gather(b, b).start()

        def outer(g, carry):
            for b in range(NBUF):
                j = g * NBUF + b
                gather(j, b).wait()
                write(j, b).start()
                jn = j + DEPTH
                bn = (b + DEPTH) % NBUF

                @pl.when(jn < groups_per_w)
                def _():
                    @pl.when(jn >= NBUF)
                    def _():
                        write(jn - NBUF, bn).wait()

                    gather(jn, bn).start()

            return carry

        lax.fori_loop(0, groups_per_w // NBUF, outer, 0)

        for b in range(NBUF):
            write(groups_per_w - NBUF + b, b).wait()

    return body(xg, table)


def kernel(x, table):
    batch, hist = x.shape
    total = batch * hist
    groups_per_w = total // (NUM_WORKERS * GROUP)
    xg = x.reshape(NUM_WORKERS, groups_per_w, GROUP).astype(jnp.int32)
    out = _embed_lookup(xg, table, groups_per_w)
    return out.reshape(batch, hist, EMBED_DIM)
